# Initial kernel scaffold; baseline (speedup 1.0000x reference)
import jax
import jax.numpy as jnp
from jax.experimental import pallas as pl


def kernel(instances, ts_w1, ts_b1, ts_w2, ts_b2, br_w1, br_b1, br_w2, br_b2, f_w1, f_b1, ln_g, ln_b, f_w2, f_b2):
    raise NotImplementedError("write your pallas kernel here")


# speedup vs baseline: 3.4589x; 3.4589x over previous
import functools

import jax
import jax.numpy as jnp
import numpy as np
from jax.experimental import pallas as pl
from jax.experimental.pallas import tpu as pltpu

B, N, D = 4, 8192, 256
H = 64
NB = 3
K = max(1, int(N * 0.1))
K5 = max(1, int(N * 0.05))
CH = 1024
NCH = N // CH

_I32_MIN = np.int32(-2147483648)
_M31 = np.int32(2147483647)


def _ordered_i32(x):
    b = jax.lax.bitcast_convert_type(x, jnp.int32)
    return b ^ ((b >> 31) & _M31)


def _ordered_to_f32(o):
    b = o ^ ((o >> 31) & _M31)
    return jax.lax.bitcast_convert_type(b, jnp.float32)


def _kth_threshold(o, k):
    t = jnp.full((o.shape[0], 1), _I32_MIN, dtype=jnp.int32)
    for bit in range(31, -1, -1):
        step = _I32_MIN if bit == 31 else np.int32(1 << bit)
        cand = t + step
        cnt = jnp.sum((o >= cand).astype(jnp.int32), axis=-1, keepdims=True)
        t = jnp.where(cnt >= k, cand, t)
    cnt_gt = jnp.sum((o > t).astype(jnp.int32), axis=-1, keepdims=True)
    return t, cnt_gt


def _body(x_ref, w1t_ref, b1_ref, w2_ref, b2_ref,
          fw1t_ref, fb1_ref, lng_ref, lnb_ref, fw2t_ref, fb2_ref,
          bag_ref, attn3_ref, avg_ref, mask_ref, ent_ref, eff_ref, t5_ref,
          ts_s, cc):
    f32 = jnp.float32

    w1t = w1t_ref[...]
    b1 = b1_ref[...]
    w2 = w2_ref[...]
    b2 = b2_ref[...]
    for b in range(B):
        for c in range(NCH):
            x = x_ref[b, c * CH:(c + 1) * CH, :]
            h = jnp.dot(x, w1t, preferred_element_type=f32) + b1
            act = jnp.concatenate(
                [jnp.maximum(h[:, :H], 0.0), jnp.tanh(h[:, H:])], axis=1)
            sc_t = jax.lax.dot_general(
                w2, act, (((0,), (1,)), ((), ())),
                preferred_element_type=f32) + b2
            ts_s[b:b + 1, c * CH:(c + 1) * CH] = sc_t[0:1, :]
            for j in range(NB):
                attn3_ref[j, b:b + 1, c * CH:(c + 1) * CH] = sc_t[j + 1:j + 2, :]

    for j in range(NB):
        asc = attn3_ref[j]
        m = jnp.max(asc, axis=-1, keepdims=True)
        e = jnp.exp(asc - m)
        z = jnp.sum(e, axis=-1, keepdims=True)
        attn3_ref[j] = e / z
    avg = (attn3_ref[0] + attn3_ref[1] + attn3_ref[2]) * (1.0 / NB)
    avg_ref[...] = avg

    ent_ref[...] = -jnp.sum(avg * jnp.log(avg + 1e-8), axis=-1, keepdims=True)
    eff_ref[...] = 1.0 / jnp.sum(avg * avg, axis=-1, keepdims=True)

    o = _ordered_i32(ts_s[...])
    t, cnt_gt = _kth_threshold(o, K)
    r = K - cnt_gt
    idx = jax.lax.broadcasted_iota(jnp.int32, (B, N), 1)
    ties = (o == t)
    jt = jnp.full((B, 1), -1, dtype=jnp.int32)
    for bit in range(12, -1, -1):
        cand = jt + np.int32(1 << bit)
        cnt = jnp.sum((ties & (idx <= cand)).astype(jnp.int32),
                      axis=-1, keepdims=True)
        jt = jnp.where(cnt <= r, cand, jt)
    mask = (o > t) | (ties & (idx <= jt))
    maskf = mask.astype(f32)
    mask_ref[...] = maskf

    oa = _ordered_i32(avg)
    t5, cnt5_gt = _kth_threshold(oa, K5)
    t5f = _ordered_to_f32(t5)
    gt_sum = jnp.sum(jnp.where(oa > t5, avg, 0.0), axis=-1, keepdims=True)
    t5_ref[...] = gt_sum + (K5 - cnt5_gt).astype(f32) * t5f

    for b in range(B):
        w5 = jnp.concatenate([
            jnp.full((1, N), 1.0 / N, dtype=f32),
            maskf[b:b + 1, :] * (1.0 / K),
            attn3_ref[0, b:b + 1, :],
            attn3_ref[1, b:b + 1, :],
            attn3_ref[2, b:b + 1, :],
        ], axis=0)
        pooled = jnp.dot(w5, x_ref[b], preferred_element_type=f32)
        for j in range(5):
            cc[b:b + 1, j * D:(j + 1) * D] = pooled[j:j + 1, :]

    fh = jnp.dot(cc[...], fw1t_ref[...], preferred_element_type=f32) + fb1_ref[...]
    mu = jnp.mean(fh, axis=-1, keepdims=True)
    dlt = fh - mu
    var = jnp.mean(dlt * dlt, axis=-1, keepdims=True)
    fh = dlt * jax.lax.rsqrt(var + 1e-5) * lng_ref[...] + lnb_ref[...]
    g = fh * 0.5 * (1.0 + jax.lax.erf(fh * np.float32(1.0 / np.sqrt(2.0))))
    bag_ref[...] = jnp.dot(g, fw2t_ref[...], preferred_element_type=f32) + fb2_ref[...]


@jax.jit
def _run(instances, w1t, b1, w2, b2, fw1t, fb1, lng, lnb, fw2t, fb2):
    f32 = jnp.float32
    outs = pl.pallas_call(
        _body,
        out_shape=[
            jax.ShapeDtypeStruct((B, 2 * D), f32),
            jax.ShapeDtypeStruct((NB, B, N), f32),
            jax.ShapeDtypeStruct((B, N), f32),
            jax.ShapeDtypeStruct((B, N), f32),
            jax.ShapeDtypeStruct((B, 1), f32),
            jax.ShapeDtypeStruct((B, 1), f32),
            jax.ShapeDtypeStruct((B, 1), f32),
        ],
        scratch_shapes=[
            pltpu.VMEM((B, N), f32),
            pltpu.VMEM((B, 5 * D), f32),
        ],
    )(instances, w1t, b1, w2, b2, fw1t, fb1, lng, lnb, fw2t, fb2)
    return outs


def kernel(instances, ts_w1, ts_b1, ts_w2, ts_b2, br_w1, br_b1, br_w2, br_b2,
           f_w1, f_b1, ln_g, ln_b, f_w2, f_b2):
    f32 = jnp.float32
    w1t = jnp.concatenate([ts_w1, br_w1.reshape(NB * H, D)], axis=0).T
    b1 = jnp.concatenate([ts_b1, br_b1.reshape(NB * H)]).reshape(1, 4 * H)
    w2 = jnp.zeros((4 * H, 4), f32)
    w2 = w2.at[:H, 0].set(ts_w2[0])
    for j in range(NB):
        w2 = w2.at[H * (j + 1):H * (j + 2), j + 1].set(br_w2[j, 0])
    b2 = jnp.concatenate([ts_b2, br_b2[:, 0]]).reshape(4, 1)

    bag, attn3, avg, maskf, ent, eff, t5 = _run(
        instances, w1t, b1, w2, b2,
        f_w1.T, f_b1.reshape(1, 2 * D), ln_g.reshape(1, 2 * D),
        ln_b.reshape(1, 2 * D), f_w2.T, f_b2.reshape(1, 2 * D))

    all_attn = jnp.transpose(attn3, (1, 0, 2))
    return (bag, all_attn, avg, maskf, ent[:, 0], eff[:, 0], t5[:, 0])
